# baseline (speedup 1.0000x reference)
import functools

import jax
import jax.numpy as jnp
from jax import lax
from jax.experimental import pallas as pl
from jax.experimental.pallas import tpu as pltpu
from jax.experimental.pallas import tpu_sc as plsc

_NUM_CORES = 2
_NUM_SUBCORES = 16
_NUM_WORKERS = _NUM_CORES * _NUM_SUBCORES

_VOCAB_BLOCK = 2048


@functools.cache
def _make_sc_gather(V, D, B):
    assert B % (8 * _NUM_WORKERS) == 0
    assert D == 16
    b_per_w = B // _NUM_WORKERS
    mesh = plsc.VectorSubcoreMesh(core_axis_name="c", subcore_axis_name="s")

    @functools.partial(
        pl.kernel,
        mesh=mesh,
        out_type=jax.ShapeDtypeStruct((D * B,), jnp.float32),
        scratch_types=[
            pltpu.VMEM((b_per_w,), jnp.int32),
            pltpu.VMEM((b_per_w, D, 128), jnp.float32),
            pltpu.VMEM((D, b_per_w), jnp.float32),
            pltpu.SemaphoreType.DMA,
        ],
        compiler_params=pltpu.CompilerParams(needs_layout_passes=False),
    )
    def gather(tabt_hbm, idx_hbm, out_hbm, idx_v, slabs_v, vals_v, sem):
        wid = lax.axis_index("s") * _NUM_CORES + lax.axis_index("c")
        base = wid * b_per_w
        pltpu.sync_copy(idx_hbm.at[pl.ds(base, b_per_w)], idx_v)
        lane_ids = lax.iota(jnp.int32, 16)
        chunks = [idx_v[pl.ds(c * 16, 16)] for c in range(b_per_w // 16)]

        def scalar_idx(j):
            sel = chunks[0]
            for c in range(1, len(chunks)):
                sel = jnp.where(j >= c * 16, chunks[c], sel)
            onehot = lane_ids == (j & 15)
            return jnp.sum(jnp.where(onehot, sel, 0))

        def fire(j, carry):
            m = pl.multiple_of((scalar_idx(j) >> 7) << 7, 128)
            pltpu.async_copy(
                tabt_hbm.at[:, pl.ds(m, 128)], slabs_v.at[j], sem
            )
            return carry

        lax.fori_loop(0, b_per_w, fire, 0, unroll=False)

        def drain(j, carry):
            pltpu.make_async_copy(
                tabt_hbm.at[:, pl.ds(0, 128)], slabs_v.at[0], sem
            ).wait()
            return carry

        lax.fori_loop(0, b_per_w, drain, 0, unroll=False)

        def extract(j, carry):
            col = scalar_idx(j) & 127
            vals = plsc.load_gather(
                slabs_v,
                [jnp.full((16,), j, jnp.int32), lane_ids,
                 jnp.full((16,), col, jnp.int32)],
            )
            plsc.store_scatter(
                vals_v, [lane_ids, jnp.full((16,), j, jnp.int32)], vals
            )
            return carry

        lax.fori_loop(0, b_per_w, extract, 0, unroll=False)
        for k in range(D):
            pltpu.sync_copy(
                vals_v.at[k], out_hbm.at[pl.ds(k * B + base, b_per_w)]
            )

    return gather


@functools.cache
def _make_projection(B, E, V):
    def body(wt_ref, b_ref, embt_ref, out_ref):
        wa = jnp.concatenate(
            [wt_ref[...], b_ref[...].reshape(1, _VOCAB_BLOCK)], axis=0
        )
        rows = [embt_ref[pl.ds(k * B, B)].reshape(1, B) for k in range(E)]
        ea = jnp.concatenate(rows + [jnp.ones((1, B), jnp.float32)], axis=0)
        out_ref[...] = lax.dot_general(
            wa,
            ea,
            dimension_numbers=(((0,), (0,)), ((), ())),
            preferred_element_type=jnp.float32,
        )

    nt = pl.cdiv(V, _VOCAB_BLOCK)
    return pl.pallas_call(
        body,
        grid=(nt,),
        in_specs=[
            pl.BlockSpec((E, _VOCAB_BLOCK), lambda i: (0, i)),
            pl.BlockSpec((_VOCAB_BLOCK,), lambda i: (i,)),
            pl.BlockSpec((E * B,), lambda i: (0,)),
        ],
        out_specs=pl.BlockSpec((_VOCAB_BLOCK, B), lambda i: (i, 0)),
        out_shape=jax.ShapeDtypeStruct((V, B), jnp.float32),
        compiler_params=pltpu.CompilerParams(
            dimension_semantics=("arbitrary",),
            vmem_limit_bytes=100 * 1024 * 1024,
        ),
    )


def kernel(center_idx, emb_table, W, b):
    idx = center_idx.astype(jnp.int32)
    V, E = emb_table.shape
    B = idx.shape[0]
    emb_t_flat = _make_sc_gather(V, E, B)(emb_table.T, idx)
    out_t = _make_projection(B, E, V)(W.T, b, emb_t_flat)
    return out_t.T
